# baseline (speedup 1.0000x reference)
import numpy as np
import jax
import jax.numpy as jnp
from jax.experimental import pallas as pl

_TEMPERATURE = 0.5
_B = 128
_V = 100000
_ROWS_PER_BLOCK = 8

_ROTS = ((13, 15, 26, 6), (17, 29, 16, 24))


def _np_threefry2x32(k0, k1, x0, x1):
    def rotl(x, d):
        return ((x << np.uint32(d)) | (x >> np.uint32(32 - d))).astype(np.uint32)

    ks = [np.uint32(k0), np.uint32(k1),
          np.uint32(k0 ^ k1 ^ np.uint32(0x1BD11BDA))]
    x0 = (x0 + ks[0]).astype(np.uint32)
    x1 = (x1 + ks[1]).astype(np.uint32)
    for g in range(5):
        for r in _ROTS[g % 2]:
            x0 = (x0 + x1).astype(np.uint32)
            x1 = (x0 ^ rotl(x1, r)).astype(np.uint32)
        x0 = (x0 + ks[(g + 1) % 3]).astype(np.uint32)
        x1 = (x1 + ks[(g + 2) % 3] + np.uint32(g + 1)).astype(np.uint32)
    return x0, x1


_FK0, _FK1 = (int(a[0]) for a in _np_threefry2x32(
    np.uint32(0), np.uint32(0), np.uint32([0]), np.uint32([12345])))
_FKS2 = _FK0 ^ _FK1 ^ 0x1BD11BDA


def _gumbel_softmax_kernel(x_ref, o_ref):
    rb, v = x_ref.shape
    blk = pl.program_id(0)

    row = jax.lax.broadcasted_iota(jnp.int32, (rb, v), 0)
    col = jax.lax.broadcasted_iota(jnp.int32, (rb, v), 1)
    lin = ((blk * rb + row) * v + col).astype(jnp.uint32)

    ks = (jnp.uint32(_FK0), jnp.uint32(_FK1), jnp.uint32(_FKS2))
    a = jnp.full((rb, v), ks[0], dtype=jnp.uint32)
    b = lin + ks[1]
    for g in range(5):
        for r in _ROTS[g % 2]:
            a = a + b
            b = a ^ ((b << r) | (b >> (32 - r)))
        a = a + ks[(g + 1) % 3]
        b = b + ks[(g + 2) % 3] + jnp.uint32(g + 1)
    bits = a ^ b

    fb = (bits >> 9) | jnp.uint32(0x3F800000)
    f = jax.lax.bitcast_convert_type(fb, jnp.float32) - jnp.float32(1.0)
    tiny = jnp.float32(np.finfo(np.float32).tiny)
    u = jnp.maximum(tiny, f * (jnp.float32(1.0) - tiny) + tiny)
    gum = -jnp.log(-jnp.log(u))

    z = (x_ref[...] + gum) * jnp.float32(1.0 / _TEMPERATURE)
    m = jnp.max(z, axis=1, keepdims=True)
    e = jnp.exp(z - m)
    s = jnp.sum(e, axis=1, keepdims=True)
    o_ref[...] = e / s


def kernel(inputs):
    return pl.pallas_call(
        _gumbel_softmax_kernel,
        grid=(_B // _ROWS_PER_BLOCK,),
        in_specs=[pl.BlockSpec((_ROWS_PER_BLOCK, _V), lambda i: (i, 0))],
        out_specs=pl.BlockSpec((_ROWS_PER_BLOCK, _V), lambda i: (i, 0)),
        out_shape=jax.ShapeDtypeStruct((_B, _V), jnp.float32),
    )(inputs)
